# baseline (speedup 1.0000x reference)
import functools

import jax
import jax.numpy as jnp
from jax import lax
from jax.experimental import pallas as pl
from jax.experimental.pallas import tpu as pltpu
from jax.experimental.pallas import tpu_sc as plsc

SEQ = 200
DIM = 128
HDIM = DIM // 2
VOCAB = 100000
BATCH = 4096
NC = 2
NS = 16
NW = NC * NS
ROWS = BATCH * SEQ
ROWS_PER_W = ROWS // NW
NCHUNK = ROWS_PER_W // SEQ
MASK = -65536


def _pack_pairs(table):
    n = table.shape[0]
    t = table.astype(jnp.bfloat16).reshape(n, 4, 2, 16).swapaxes(2, 3)
    return jax.lax.bitcast_convert_type(t, jnp.int32).reshape(n, HDIM)


def _emb_body(idx_hbm, tok_hbm, pos_hbm, out_hbm,
              g0, g1, o0, o1, i0, i1, pos_v,
              gs0, gs1, ss0, ss1, is0, is1):
    G = [g0, g1]
    O = [o0, o1]
    I = [i0, i1]
    GS = [gs0, gs1]
    SS = [ss0, ss1]
    IS = [is0, is1]

    wid = lax.axis_index("s") * NC + lax.axis_index("c")
    base = wid * ROWS_PER_W

    def idx_issue(c, slot):
        pltpu.async_copy(idx_hbm.at[pl.ds(base + c * SEQ, SEQ)],
                         I[slot], IS[slot])

    def idx_wait(slot):
        pltpu.make_async_copy(idx_hbm.at[pl.ds(0, SEQ)],
                              I[slot], IS[slot]).wait()

    def gather_issue(slot):
        pltpu.async_copy(tok_hbm.at[I[slot]], G[slot], GS[slot])

    def gather_wait(slot):
        pltpu.make_async_copy(tok_hbm.at[pl.ds(0, SEQ)],
                              G[slot], GS[slot]).wait()

    def store_issue(c, slot):
        pltpu.async_copy(O[slot], out_hbm.at[pl.ds(base + c * SEQ, SEQ)],
                         SS[slot])

    def store_wait(slot):
        pltpu.make_async_copy(O[slot], out_hbm.at[pl.ds(0, SEQ)],
                              SS[slot]).wait()

    def add_chunk(slot):
        gbuf = G[slot]
        obuf = O[slot]

        @plsc.parallel_loop(0, SEQ, 1, unroll=4)
        def _(p):
            for g in range(4):
                w = gbuf[p, pl.ds(g * 16, 16)]
                q = pos_v[p, pl.ds(g * 16, 16)]
                lo = lax.bitcast_convert_type(lax.shift_left(w, 16), jnp.float32)
                hi = lax.bitcast_convert_type(jnp.bitwise_and(w, MASK), jnp.float32)
                qlo = lax.bitcast_convert_type(lax.shift_left(q, 16), jnp.float32)
                qhi = lax.bitcast_convert_type(jnp.bitwise_and(q, MASK), jnp.float32)
                obuf[p, pl.ds(g * 32, 16)] = lo + qlo
                obuf[p, pl.ds(g * 32 + 16, 16)] = hi + qhi

    def substep(c, b, do_idx, do_gather, do_store_wait):
        nb = 1 - b
        gather_wait(b)
        if do_idx:
            idx_issue(c + 2, b)
        if do_gather:
            idx_wait(nb)
            gather_issue(nb)
        if do_store_wait:
            store_wait(b)
        add_chunk(b)
        store_issue(c, b)

    pltpu.sync_copy(pos_hbm, pos_v)
    idx_issue(0, 0)
    idx_issue(1, 1)
    idx_wait(0)
    gather_issue(0)
    substep(0, 0, True, True, False)
    substep(1, 1, True, True, False)

    def outer(t, carry):
        substep(2 * t, 0, True, True, True)
        substep(2 * t + 1, 1, True, True, True)
        return carry

    lax.fori_loop(1, 63, outer, 0, unroll=False)

    substep(126, 0, False, True, True)
    substep(127, 1, False, False, True)
    store_wait(0)
    store_wait(1)


def kernel(inputs, token_table, pos_table):
    idx_flat = inputs.reshape(ROWS).astype(jnp.int32)
    tok_packed = _pack_pairs(token_table)
    pos_packed = _pack_pairs(pos_table)
    mesh = plsc.VectorSubcoreMesh(core_axis_name="c", subcore_axis_name="s")
    k = functools.partial(
        pl.kernel,
        out_type=jax.ShapeDtypeStruct((ROWS, DIM), jnp.float32),
        mesh=mesh,
        compiler_params=pltpu.CompilerParams(use_tc_tiling_on_sc=False),
        scratch_types=(
            [pltpu.VMEM((SEQ, HDIM), jnp.int32) for _ in range(2)]
            + [pltpu.VMEM((SEQ, DIM), jnp.float32) for _ in range(2)]
            + [pltpu.VMEM((SEQ,), jnp.int32) for _ in range(2)]
            + [pltpu.VMEM((SEQ, HDIM), jnp.int32)]
            + [pltpu.SemaphoreType.DMA for _ in range(6)]
        ),
    )(_emb_body)
    out = k(idx_flat, tok_packed, pos_packed)
    return out.reshape(BATCH, SEQ, DIM)


# speedup vs baseline: 1.0052x; 1.0052x over previous
import functools

import jax
import jax.numpy as jnp
from jax import lax
from jax.experimental import pallas as pl
from jax.experimental.pallas import tpu as pltpu
from jax.experimental.pallas import tpu_sc as plsc

SEQ = 200
DIM = 128
HDIM = DIM // 2
VOCAB = 100000
BATCH = 4096
NC = 2
NS = 16
NW = NC * NS
ROWS = BATCH * SEQ
ROWS_PER_W = ROWS // NW
NCHUNK = ROWS_PER_W // SEQ
MASK = -65536


def _pack_pairs(table):
    n = table.shape[0]
    t = table.reshape(n, 4, 2, 16).swapaxes(2, 3).astype(jnp.bfloat16)
    return jax.lax.bitcast_convert_type(t, jnp.int32).reshape(n, HDIM)


def _emb_body(idx_hbm, tok_hbm, pos_hbm, out_hbm,
              g0, g1, o0, o1, i0, i1, pos_v,
              gs0, gs1, ss0, ss1, is0, is1):
    G = [g0, g1]
    O = [o0, o1]
    I = [i0, i1]
    GS = [gs0, gs1]
    SS = [ss0, ss1]
    IS = [is0, is1]

    wid = lax.axis_index("s") * NC + lax.axis_index("c")
    base = wid * ROWS_PER_W

    def idx_issue(c, slot):
        pltpu.async_copy(idx_hbm.at[pl.ds(base + c * SEQ, SEQ)],
                         I[slot], IS[slot])

    def idx_wait(slot):
        pltpu.make_async_copy(idx_hbm.at[pl.ds(0, SEQ)],
                              I[slot], IS[slot]).wait()

    def gather_issue(slot):
        pltpu.async_copy(tok_hbm.at[I[slot]], G[slot], GS[slot])

    def gather_wait(slot):
        pltpu.make_async_copy(tok_hbm.at[pl.ds(0, SEQ)],
                              G[slot], GS[slot]).wait()

    def store_issue(c, slot):
        pltpu.async_copy(O[slot], out_hbm.at[pl.ds(base + c * SEQ, SEQ)],
                         SS[slot])

    def store_wait(slot):
        pltpu.make_async_copy(O[slot], out_hbm.at[pl.ds(0, SEQ)],
                              SS[slot]).wait()

    def add_chunk(slot):
        gbuf = G[slot]
        obuf = O[slot]

        @plsc.parallel_loop(0, SEQ, 1, unroll=4)
        def _(p):
            for g in range(4):
                w = gbuf[p, pl.ds(g * 16, 16)]
                q = pos_v[p, pl.ds(g * 16, 16)]
                lo = lax.bitcast_convert_type(lax.shift_left(w, 16), jnp.float32)
                hi = lax.bitcast_convert_type(jnp.bitwise_and(w, MASK), jnp.float32)
                qlo = lax.bitcast_convert_type(lax.shift_left(q, 16), jnp.float32)
                qhi = lax.bitcast_convert_type(jnp.bitwise_and(q, MASK), jnp.float32)
                obuf[p, pl.ds(g * 32, 16)] = lo + qlo
                obuf[p, pl.ds(g * 32 + 16, 16)] = hi + qhi

    def substep(c, b, do_idx, do_gather, do_store_wait):
        nb = 1 - b
        gather_wait(b)
        if do_idx:
            idx_issue(c + 2, b)
        if do_gather:
            idx_wait(nb)
            gather_issue(nb)
        if do_store_wait:
            store_wait(b)
        add_chunk(b)
        store_issue(c, b)

    pltpu.sync_copy(pos_hbm, pos_v)
    idx_issue(0, 0)
    idx_issue(1, 1)
    idx_wait(0)
    gather_issue(0)
    substep(0, 0, True, True, False)
    substep(1, 1, True, True, False)

    def outer(t, carry):
        substep(2 * t, 0, True, True, True)
        substep(2 * t + 1, 1, True, True, True)
        return carry

    lax.fori_loop(1, 63, outer, 0, unroll=False)

    substep(126, 0, False, True, True)
    substep(127, 1, False, False, True)
    store_wait(0)
    store_wait(1)


def kernel(inputs, token_table, pos_table):
    idx_flat = inputs.reshape(ROWS).astype(jnp.int32)
    tok_packed = _pack_pairs(token_table)
    pos_packed = _pack_pairs(pos_table)
    mesh = plsc.VectorSubcoreMesh(core_axis_name="c", subcore_axis_name="s")
    k = functools.partial(
        pl.kernel,
        out_type=jax.ShapeDtypeStruct((ROWS, DIM), jnp.float32),
        mesh=mesh,
        compiler_params=pltpu.CompilerParams(use_tc_tiling_on_sc=False),
        scratch_types=(
            [pltpu.VMEM((SEQ, HDIM), jnp.int32) for _ in range(2)]
            + [pltpu.VMEM((SEQ, DIM), jnp.float32) for _ in range(2)]
            + [pltpu.VMEM((SEQ,), jnp.int32) for _ in range(2)]
            + [pltpu.VMEM((SEQ, HDIM), jnp.int32)]
            + [pltpu.SemaphoreType.DMA for _ in range(6)]
        ),
    )(_emb_body)
    out = k(idx_flat, tok_packed, pos_packed)
    return out.reshape(BATCH, SEQ, DIM)


# speedup vs baseline: 1.2860x; 1.2794x over previous
import functools

import jax
import jax.numpy as jnp
from jax import lax
from jax.experimental import pallas as pl
from jax.experimental.pallas import tpu as pltpu
from jax.experimental.pallas import tpu_sc as plsc

SEQ = 200
DIM = 128
BATCH = 4096
NC = 2
NS = 16
NW = NC * NS
ROWS = BATCH * SEQ
ROWS_PER_W = ROWS // NW
NCHUNK = ROWS_PER_W // SEQ
NBUF = 3


def _emb_body(idx_hbm, tok_hbm, pos_hbm, out_hbm,
              g0, g1, g2, idx_v, pos_v,
              gs0, gs1, gs2, ss0, ss1, ss2):
    G = [g0, g1, g2]
    GS = [gs0, gs1, gs2]
    SS = [ss0, ss1, ss2]

    wid = lax.axis_index("s") * NC + lax.axis_index("c")
    base = wid * ROWS_PER_W

    def gather_issue(c, slot):
        pltpu.async_copy(tok_hbm.at[idx_v.at[pl.ds(c * SEQ, SEQ)]],
                         G[slot], GS[slot])

    def gather_wait(slot):
        pltpu.make_async_copy(tok_hbm.at[pl.ds(0, SEQ)],
                              G[slot], GS[slot]).wait()

    def store_issue(c, slot):
        pltpu.async_copy(G[slot], out_hbm.at[pl.ds(base + c * SEQ, SEQ)],
                         SS[slot])

    def store_wait(slot):
        pltpu.make_async_copy(G[slot], out_hbm.at[pl.ds(0, SEQ)],
                              SS[slot]).wait()

    def add_chunk(slot):
        gbuf = G[slot]

        @plsc.parallel_loop(0, SEQ, 1, unroll=4)
        def _(p):
            for d in range(DIM // 16):
                sl = pl.ds(d * 16, 16)
                gbuf[p, sl] = gbuf[p, sl] + pos_v[p, sl]

    def substep(c, slot, do_store_wait, do_next_gather):
        nb = (slot + 1) % NBUF
        gather_wait(slot)
        if do_next_gather:
            if do_store_wait:
                store_wait(nb)
            gather_issue(c + 1, nb)
        add_chunk(slot)
        store_issue(c, slot)

    pltpu.sync_copy(idx_hbm.at[pl.ds(base, ROWS_PER_W)], idx_v)
    pltpu.sync_copy(pos_hbm, pos_v)
    gather_issue(0, 0)
    substep(0, 0, False, True)
    substep(1, 1, False, True)
    substep(2, 2, True, True)

    def outer(t, carry):
        for b in range(NBUF):
            substep(t * NBUF + b, b, True, True)
        return carry

    lax.fori_loop(1, 40, outer, 0, unroll=False)

    for c in range(120, NCHUNK):
        substep(c, c % NBUF, True, c + 1 < NCHUNK)
    store_wait((NCHUNK - 2) % NBUF)
    store_wait((NCHUNK - 1) % NBUF)


def kernel(inputs, token_table, pos_table):
    idx_flat = inputs.reshape(ROWS).astype(jnp.int32)
    mesh = plsc.VectorSubcoreMesh(core_axis_name="c", subcore_axis_name="s")
    k = functools.partial(
        pl.kernel,
        out_type=jax.ShapeDtypeStruct((ROWS, DIM), jnp.float32),
        mesh=mesh,
        scratch_types=(
            [pltpu.VMEM((SEQ, DIM), jnp.float32) for _ in range(NBUF)]
            + [pltpu.VMEM((ROWS_PER_W,), jnp.int32)]
            + [pltpu.VMEM((SEQ, DIM), jnp.float32)]
            + [pltpu.SemaphoreType.DMA for _ in range(2 * NBUF)]
        ),
    )(_emb_body)
    out = k(idx_flat, token_table, pos_table)
    return out.reshape(BATCH, SEQ, DIM)


# speedup vs baseline: 1.3115x; 1.0198x over previous
import functools

import jax
import jax.numpy as jnp
from jax import lax
from jax.experimental import pallas as pl
from jax.experimental.pallas import tpu as pltpu
from jax.experimental.pallas import tpu_sc as plsc

SEQ = 200
DIM = 128
BATCH = 4096
NC = 2
NS = 16
NW = NC * NS
ROWS = BATCH * SEQ
ROWS_PER_W = ROWS // NW
NCHUNK = ROWS_PER_W // SEQ
NBUF = 4


def _emb_body(idx_hbm, tok_hbm, pos_hbm, out_hbm,
              g0, g1, g2, g3, i0, i1, i2, i3, pos_v,
              gs0, gs1, gs2, gs3, is0, is1, is2, is3,
              ss0, ss1, ss2, ss3):
    G = [g0, g1, g2, g3]
    I = [i0, i1, i2, i3]
    GS = [gs0, gs1, gs2, gs3]
    IS = [is0, is1, is2, is3]
    SS = [ss0, ss1, ss2, ss3]

    wid = lax.axis_index("s") * NC + lax.axis_index("c")
    base = wid * ROWS_PER_W

    def idx_issue(c, slot):
        pltpu.async_copy(idx_hbm.at[pl.ds(base + c * SEQ, SEQ)],
                         I[slot], IS[slot])

    def idx_wait(slot):
        pltpu.make_async_copy(idx_hbm.at[pl.ds(0, SEQ)],
                              I[slot], IS[slot]).wait()

    def gather_issue(slot):
        pltpu.async_copy(tok_hbm.at[I[slot]], G[slot], GS[slot])

    def gather_wait(slot):
        pltpu.make_async_copy(tok_hbm.at[pl.ds(0, SEQ)],
                              G[slot], GS[slot]).wait()

    def store_issue(c, slot):
        pltpu.async_copy(G[slot], out_hbm.at[pl.ds(base + c * SEQ, SEQ)],
                         SS[slot])

    def store_wait(slot):
        pltpu.make_async_copy(G[slot], out_hbm.at[pl.ds(0, SEQ)],
                              SS[slot]).wait()

    def add_chunk(slot):
        gbuf = G[slot]

        @plsc.parallel_loop(0, SEQ, 1, unroll=4)
        def _(p):
            for d in range(DIM // 16):
                sl = pl.ds(d * 16, 16)
                gbuf[p, sl] = gbuf[p, sl] + pos_v[p, sl]

    def substep(c, b, do_idx, do_gather2, do_store_wait):
        b2 = (b + 2) % NBUF
        gather_wait(b)
        if do_idx:
            idx_issue(c + NBUF, b)
        if do_gather2:
            if do_store_wait:
                store_wait(b2)
            idx_wait(b2)
            gather_issue(b2)
        add_chunk(b)
        store_issue(c, b)

    pltpu.sync_copy(pos_hbm, pos_v)
    for b in range(NBUF):
        idx_issue(b, b)
    idx_wait(0)
    gather_issue(0)
    idx_wait(1)
    gather_issue(1)
    substep(0, 0, True, True, False)
    substep(1, 1, True, True, False)
    substep(2, 2, True, True, True)
    substep(3, 3, True, True, True)

    def outer(t, carry):
        for b in range(NBUF):
            substep(t * NBUF + b, b, True, True, True)
        return carry

    lax.fori_loop(1, 31, outer, 0, unroll=False)

    for c in range(124, NCHUNK):
        substep(c, c % NBUF, False, c + 2 < NCHUNK, c + 2 < NCHUNK)
    for b in range(NBUF):
        store_wait(b)


def kernel(inputs, token_table, pos_table):
    idx_flat = inputs.reshape(ROWS).astype(jnp.int32)
    mesh = plsc.VectorSubcoreMesh(core_axis_name="c", subcore_axis_name="s")
    k = functools.partial(
        pl.kernel,
        out_type=jax.ShapeDtypeStruct((ROWS, DIM), jnp.float32),
        mesh=mesh,
        scratch_types=(
            [pltpu.VMEM((SEQ, DIM), jnp.float32) for _ in range(NBUF)]
            + [pltpu.VMEM((SEQ,), jnp.int32) for _ in range(NBUF)]
            + [pltpu.VMEM((SEQ, DIM), jnp.float32)]
            + [pltpu.SemaphoreType.DMA for _ in range(3 * NBUF)]
        ),
    )(_emb_body)
    out = k(idx_flat, token_table, pos_table)
    return out.reshape(BATCH, SEQ, DIM)
